# baseline (speedup 1.0000x reference)
import math

import jax
import jax.numpy as jnp
from jax.experimental import pallas as pl
from jax.experimental.pallas import tpu as pltpu

B, H, W = 32, 32, 32
HW = H * W
G_RATIO = 0.5
TMAX = 0.25
TB_FACTOR = 0.001
T = int(TMAX * H * W)
NSPLIT = 2
NR = B // NSPLIT


def _heuristic_and_goal(goal_maps):
    b, h, w = goal_maps.shape
    ys, xs = jnp.meshgrid(jnp.arange(h), jnp.arange(w), indexing='ij')
    loc = jnp.stack([ys, xs], axis=0).astype(jnp.float32)
    goal_idx = jnp.argmax(goal_maps.reshape(b, -1), axis=-1)
    gy = (goal_idx // w).astype(jnp.float32)
    gx = (goal_idx % w).astype(jnp.float32)
    goal_loc = jnp.stack([gy, gx], axis=1)[:, :, None, None]
    dxdy = jnp.abs(loc[None] - goal_loc)
    hmap = dxdy.sum(axis=1) - dxdy.min(axis=1)
    euc = jnp.sqrt((dxdy ** 2).sum(axis=1))
    return hmap + TB_FACTOR * euc, goal_idx


def _astar_body(hm_ref, cost_ref, start_ref, gidx_ref, iota_ref, r_ref, c_ref,
                out_ref, g_ref, open_ref, hist_ref):
    sl = [slice(k * NR, (k + 1) * NR) for k in range(NSPLIT)]
    g_ref[...] = jnp.zeros((B, HW), jnp.float32)
    hist_ref[...] = jnp.zeros((B, HW), jnp.float32)
    open_ref[...] = start_ref[...]

    def step(_, tok):
        gs = [g_ref[s, :] for s in sl]
        opens = [open_ref[s, :] for s in sl]

        vs, ss = [], []
        for k in range(NSPLIT):
            f = G_RATIO * gs[k] + (1.0 - G_RATIO) * hm_ref[sl[k], :]
            v = jnp.exp(-1.0 * f / math.sqrt(W)) * opens[k]
            vs.append(v)
            ss.append(jnp.maximum(jnp.sum(v, axis=-1, keepdims=True), 1e-30))

        ys = [vs[k] / ss[k] for k in range(NSPLIT)]

        sel_idxs = []
        for k in range(NSPLIT):
            sel_idxs.append(
                jnp.argmax(ys[k], axis=-1).astype(jnp.int32).reshape(NR, 1))

        for k in range(NSPLIT):
            s = sl[k]
            g = gs[k]
            open_m = opens[k]
            hist = hist_ref[s, :]
            iota = iota_ref[s, :]
            sel_idx = sel_idxs[k]
            sel_mask = iota == sel_idx
            unsolved = sel_idx != gidx_ref[s, :]
            g2v = jnp.sum(jnp.where(sel_mask, g + cost_ref[s, :], 0.0),
                          axis=-1, keepdims=True)
            rm = (sel_idx >> 5) - 1
            cm = (sel_idx & 31) - 1
            window = (((r_ref[s, :] - rm).astype(jnp.uint32) <= 2)
                      & ((c_ref[s, :] - cm).astype(jnp.uint32) <= 2))
            newly = window & (((open_m + hist) == 0.0)
                              | ((open_m > 0.0) & (g > g2v)))
            g_ref[s, :] = jnp.where(newly, g2v, g)
            open_ref[s, :] = jnp.where(
                newly, 1.0, jnp.where(sel_mask & unsolved, 0.0, open_m))
            hist_ref[s, :] = jnp.where(sel_mask, 1.0, hist)
        return tok

    def step4(i, tok):
        return step(i, step(i, step(i, step(i, tok))))

    jax.lax.fori_loop(0, T // 4, step4, 0)
    out_ref[...] = hist_ref[...]


def kernel(map_designs, start_maps, goal_maps):
    hmap, goal_idx = _heuristic_and_goal(goal_maps)
    hm = (hmap + map_designs).reshape(B, HW)
    cost = map_designs.reshape(B, HW)
    start = start_maps.reshape(B, HW)
    gidx = goal_idx.astype(jnp.int32).reshape(B, 1)
    iota = jax.lax.broadcasted_iota(jnp.int32, (B, HW), 1)
    r_cell = iota >> 5
    c_cell = iota & 31
    hist = pl.pallas_call(
        _astar_body,
        out_shape=jax.ShapeDtypeStruct((B, HW), jnp.float32),
        scratch_shapes=[pltpu.VMEM((B, HW), jnp.float32)] * 3,
    )(hm, cost, start, gidx, iota, r_cell, c_cell)
    return hist.reshape(B, H, W)


# speedup vs baseline: 1.0296x; 1.0296x over previous
import math

import jax
import jax.numpy as jnp
from jax.experimental import pallas as pl
from jax.experimental.pallas import tpu as pltpu

B, H, W = 32, 32, 32
HW = H * W
G_RATIO = 0.5
TMAX = 0.25
TB_FACTOR = 0.001
T = int(TMAX * H * W)
NSPLIT = 2
NR = B // NSPLIT


def _heuristic_and_goal(goal_maps):
    b, h, w = goal_maps.shape
    ys, xs = jnp.meshgrid(jnp.arange(h), jnp.arange(w), indexing='ij')
    loc = jnp.stack([ys, xs], axis=0).astype(jnp.float32)
    goal_idx = jnp.argmax(goal_maps.reshape(b, -1), axis=-1)
    gy = (goal_idx // w).astype(jnp.float32)
    gx = (goal_idx % w).astype(jnp.float32)
    goal_loc = jnp.stack([gy, gx], axis=1)[:, :, None, None]
    dxdy = jnp.abs(loc[None] - goal_loc)
    hmap = dxdy.sum(axis=1) - dxdy.min(axis=1)
    euc = jnp.sqrt((dxdy ** 2).sum(axis=1))
    return hmap + TB_FACTOR * euc, goal_idx


def _astar_body(hm_ref, cost_ref, start_ref, gidx_ref, iota_ref, r_ref, c_ref,
                out_ref, g_ref, open_ref, hist_ref):
    sl = [slice(k * NR, (k + 1) * NR) for k in range(NSPLIT)]
    g_ref[...] = jnp.zeros((B, HW), jnp.float32)
    hist_ref[...] = jnp.zeros((B, HW), jnp.float32)
    open_ref[...] = start_ref[...]

    def step(_, tok):
        gs = [g_ref[s, :] for s in sl]
        opens = [open_ref[s, :] for s in sl]

        vs, ss = [], []
        for k in range(NSPLIT):
            f = G_RATIO * gs[k] + (1.0 - G_RATIO) * hm_ref[sl[k], :]
            v = jnp.exp(-1.0 * f / math.sqrt(W)) * opens[k]
            vs.append(v)
            ss.append(jnp.maximum(jnp.sum(v, axis=-1, keepdims=True), 1e-30))

        ys = [vs[k] / ss[k] for k in range(NSPLIT)]

        sel_idxs = []
        for k in range(NSPLIT):
            sel_idxs.append(
                jnp.argmax(ys[k], axis=-1).astype(jnp.int32).reshape(NR, 1))

        for k in range(NSPLIT):
            s = sl[k]
            g = gs[k]
            open_m = opens[k]
            hist = hist_ref[s, :]
            iota = iota_ref[s, :]
            sel_idx = sel_idxs[k]
            sel_mask = iota == sel_idx
            unsolved = sel_idx != gidx_ref[s, :]
            g2v = jnp.sum(jnp.where(sel_mask, g + cost_ref[s, :], 0.0),
                          axis=-1, keepdims=True)
            rm = (sel_idx >> 5) - 1
            cm = (sel_idx & 31) - 1
            window = (((r_ref[s, :] - rm).astype(jnp.uint32) <= 2)
                      & ((c_ref[s, :] - cm).astype(jnp.uint32) <= 2))
            newly = window & (((open_m + hist) == 0.0)
                              | ((open_m > 0.0) & (g > g2v)))
            g_ref[s, :] = jnp.where(newly, g2v, g)
            open_ref[s, :] = jnp.where(
                newly, 1.0, jnp.where(sel_mask & unsolved, 0.0, open_m))
            hist_ref[s, :] = jnp.where(sel_mask, 1.0, hist)
        return tok

    def step8(i, tok):
        for _ in range(8):
            tok = step(i, tok)
        return tok

    jax.lax.fori_loop(0, T // 8, step8, 0)
    out_ref[...] = hist_ref[...]


def kernel(map_designs, start_maps, goal_maps):
    hmap, goal_idx = _heuristic_and_goal(goal_maps)
    hm = (hmap + map_designs).reshape(B, HW)
    cost = map_designs.reshape(B, HW)
    start = start_maps.reshape(B, HW)
    gidx = goal_idx.astype(jnp.int32).reshape(B, 1)
    iota = jax.lax.broadcasted_iota(jnp.int32, (B, HW), 1)
    r_cell = iota >> 5
    c_cell = iota & 31
    hist = pl.pallas_call(
        _astar_body,
        out_shape=jax.ShapeDtypeStruct((B, HW), jnp.float32),
        scratch_shapes=[pltpu.VMEM((B, HW), jnp.float32)] * 3,
    )(hm, cost, start, gidx, iota, r_cell, c_cell)
    return hist.reshape(B, H, W)


# speedup vs baseline: 1.0431x; 1.0132x over previous
import math

import jax
import jax.numpy as jnp
from jax.experimental import pallas as pl
from jax.experimental.pallas import tpu as pltpu

B, H, W = 32, 32, 32
HW = H * W
G_RATIO = 0.5
TMAX = 0.25
TB_FACTOR = 0.001
T = int(TMAX * H * W)
NSPLIT = 2
NR = B // NSPLIT


def _heuristic_and_goal(goal_maps):
    b, h, w = goal_maps.shape
    ys, xs = jnp.meshgrid(jnp.arange(h), jnp.arange(w), indexing='ij')
    loc = jnp.stack([ys, xs], axis=0).astype(jnp.float32)
    goal_idx = jnp.argmax(goal_maps.reshape(b, -1), axis=-1)
    gy = (goal_idx // w).astype(jnp.float32)
    gx = (goal_idx % w).astype(jnp.float32)
    goal_loc = jnp.stack([gy, gx], axis=1)[:, :, None, None]
    dxdy = jnp.abs(loc[None] - goal_loc)
    hmap = dxdy.sum(axis=1) - dxdy.min(axis=1)
    euc = jnp.sqrt((dxdy ** 2).sum(axis=1))
    return hmap + TB_FACTOR * euc, goal_idx


def _astar_body(hm_ref, cost_ref, start_ref, gidx_ref, iota_ref, r_ref, c_ref,
                out_ref, g_ref, open_ref, hist_ref):
    sl = [slice(k * NR, (k + 1) * NR) for k in range(NSPLIT)]
    g_ref[...] = jnp.zeros((B, HW), jnp.float32)
    hist_ref[...] = jnp.zeros((B, HW), jnp.float32)
    open_ref[...] = start_ref[...]

    def step(_, tok):
        gs = [g_ref[s, :] for s in sl]
        opens = [open_ref[s, :] for s in sl]

        vs, ss = [], []
        for k in range(NSPLIT):
            f = G_RATIO * gs[k] + (1.0 - G_RATIO) * hm_ref[sl[k], :]
            v = jnp.exp(-1.0 * f / math.sqrt(W)) * opens[k]
            vs.append(v)
            ss.append(jnp.maximum(jnp.sum(v, axis=-1, keepdims=True), 1e-30))

        ys = [vs[k] / ss[k] for k in range(NSPLIT)]

        sel_idxs = []
        for k in range(NSPLIT):
            sel_idxs.append(
                jnp.argmax(ys[k], axis=-1).astype(jnp.int32).reshape(NR, 1))

        for k in range(NSPLIT):
            s = sl[k]
            g = gs[k]
            open_m = opens[k]
            hist = hist_ref[s, :]
            iota = iota_ref[s, :]
            sel_idx = sel_idxs[k]
            sel_mask = iota == sel_idx
            unsolved = sel_idx != gidx_ref[s, :]
            g2v = jnp.sum(jnp.where(sel_mask, g + cost_ref[s, :], 0.0),
                          axis=-1, keepdims=True)
            rm = (sel_idx >> 5) - 1
            cm = (sel_idx & 31) - 1
            window = (((r_ref[s, :] - rm).astype(jnp.uint32) <= 2)
                      & ((c_ref[s, :] - cm).astype(jnp.uint32) <= 2))
            newly = window & (((open_m + hist) == 0.0)
                              | ((open_m > 0.0) & (g > g2v)))
            g_ref[s, :] = jnp.where(newly, g2v, g)
            open_ref[s, :] = jnp.where(
                newly, 1.0, jnp.where(sel_mask & unsolved, 0.0, open_m))
            hist_ref[s, :] = jnp.where(sel_mask, 1.0, hist)
        return tok

    def step8(i, tok):
        for _ in range(16):
            tok = step(i, tok)
        return tok

    jax.lax.fori_loop(0, T // 16, step8, 0)
    out_ref[...] = hist_ref[...]


def kernel(map_designs, start_maps, goal_maps):
    hmap, goal_idx = _heuristic_and_goal(goal_maps)
    hm = (hmap + map_designs).reshape(B, HW)
    cost = map_designs.reshape(B, HW)
    start = start_maps.reshape(B, HW)
    gidx = goal_idx.astype(jnp.int32).reshape(B, 1)
    iota = jax.lax.broadcasted_iota(jnp.int32, (B, HW), 1)
    r_cell = iota >> 5
    c_cell = iota & 31
    hist = pl.pallas_call(
        _astar_body,
        out_shape=jax.ShapeDtypeStruct((B, HW), jnp.float32),
        scratch_shapes=[pltpu.VMEM((B, HW), jnp.float32)] * 3,
    )(hm, cost, start, gidx, iota, r_cell, c_cell)
    return hist.reshape(B, H, W)


# speedup vs baseline: 1.0434x; 1.0003x over previous
import math

import jax
import jax.numpy as jnp
from jax.experimental import pallas as pl
from jax.experimental.pallas import tpu as pltpu

B, H, W = 32, 32, 32
HW = H * W
G_RATIO = 0.5
TMAX = 0.25
TB_FACTOR = 0.001
T = int(TMAX * H * W)
NSPLIT = 2
NR = B // NSPLIT


def _heuristic_and_goal(goal_maps):
    b, h, w = goal_maps.shape
    ys, xs = jnp.meshgrid(jnp.arange(h), jnp.arange(w), indexing='ij')
    loc = jnp.stack([ys, xs], axis=0).astype(jnp.float32)
    goal_idx = jnp.argmax(goal_maps.reshape(b, -1), axis=-1)
    gy = (goal_idx // w).astype(jnp.float32)
    gx = (goal_idx % w).astype(jnp.float32)
    goal_loc = jnp.stack([gy, gx], axis=1)[:, :, None, None]
    dxdy = jnp.abs(loc[None] - goal_loc)
    hmap = dxdy.sum(axis=1) - dxdy.min(axis=1)
    euc = jnp.sqrt((dxdy ** 2).sum(axis=1))
    return hmap + TB_FACTOR * euc, goal_idx


def _astar_body(hm_ref, cost_ref, start_ref, gidx_ref, iota_ref, r_ref, c_ref,
                out_ref, g_ref, open_ref, hist_ref):
    sl = [slice(k * NR, (k + 1) * NR) for k in range(NSPLIT)]
    g_ref[...] = jnp.zeros((B, HW), jnp.float32)
    hist_ref[...] = jnp.zeros((B, HW), jnp.float32)
    open_ref[...] = start_ref[...]

    def step(_, tok):
        gs = [g_ref[s, :] for s in sl]
        opens = [open_ref[s, :] for s in sl]

        vs, ss = [], []
        for k in range(NSPLIT):
            f = G_RATIO * gs[k] + (1.0 - G_RATIO) * hm_ref[sl[k], :]
            v = jnp.exp(-1.0 * f / math.sqrt(W)) * opens[k]
            vs.append(v)
            ss.append(jnp.maximum(jnp.sum(v, axis=-1, keepdims=True), 1e-30))

        ys = [vs[k] / ss[k] for k in range(NSPLIT)]

        sel_idxs = []
        for k in range(NSPLIT):
            sel_idxs.append(
                jnp.argmax(ys[k], axis=-1).astype(jnp.int32).reshape(NR, 1))

        for k in range(NSPLIT):
            s = sl[k]
            g = gs[k]
            open_m = opens[k]
            hist = hist_ref[s, :]
            iota = iota_ref[s, :]
            sel_idx = sel_idxs[k]
            sel_mask = iota == sel_idx
            unsolved = sel_idx != gidx_ref[s, :]
            g2v = jnp.sum(jnp.where(sel_mask, g + cost_ref[s, :], 0.0),
                          axis=-1, keepdims=True)
            rm = (sel_idx >> 5) - 1
            cm = (sel_idx & 31) - 1
            window = (((r_ref[s, :] - rm).astype(jnp.uint32) <= 2)
                      & ((c_ref[s, :] - cm).astype(jnp.uint32) <= 2))
            newly = window & (((open_m + hist) == 0.0)
                              | ((open_m > 0.0) & (g > g2v)))
            g_ref[s, :] = jnp.where(newly, g2v, g)
            open_ref[s, :] = jnp.where(
                newly, 1.0, jnp.where(sel_mask & unsolved, 0.0, open_m))
            hist_ref[s, :] = jnp.where(sel_mask, 1.0, hist)
        return tok

    UNROLL = 16
    def step_block(i, tok):
        for _ in range(UNROLL):
            tok = step(i, tok)
        return tok

    jax.lax.fori_loop(0, T // UNROLL, step_block, 0)
    out_ref[...] = hist_ref[...]


def kernel(map_designs, start_maps, goal_maps):
    hmap, goal_idx = _heuristic_and_goal(goal_maps)
    hm = (hmap + map_designs).reshape(B, HW)
    cost = map_designs.reshape(B, HW)
    start = start_maps.reshape(B, HW)
    gidx = goal_idx.astype(jnp.int32).reshape(B, 1)
    iota = jax.lax.broadcasted_iota(jnp.int32, (B, HW), 1)
    r_cell = iota >> 5
    c_cell = iota & 31
    hist = pl.pallas_call(
        _astar_body,
        out_shape=jax.ShapeDtypeStruct((B, HW), jnp.float32),
        scratch_shapes=[pltpu.VMEM((B, HW), jnp.float32)] * 3,
    )(hm, cost, start, gidx, iota, r_cell, c_cell)
    return hist.reshape(B, H, W)
